# baseline (speedup 1.0000x reference)
import functools

import jax
import jax.numpy as jnp
from jax import lax
from jax.experimental import pallas as pl
from jax.experimental.pallas import tpu as pltpu
from jax.experimental.pallas import tpu_sc as plsc

E = 16
TOP_K = 2
D = 768
DFF = 768
T = 2048
C = 512
S = E * C
S_PAD = S + 8
ALPHA = 1.702
LIMIT = 7.0

NC, NS = 2, 16
NW = NC * NS
TPW = T // NW


BK = 512


def _router_body(x_ref, wr_ref, br_ref, gp_ref, w0_ref, w1_ref, carry_ref):
    b = pl.program_id(0)

    @pl.when(b == 0)
    def _():
        carry_ref[...] = jnp.zeros((1, E), jnp.float32)

    logits = lax.dot_general(x_ref[...], wr_ref[...], (((1,), (1,)), ((), ())),
                             preferred_element_type=jnp.float32) + br_ref[...]
    m = jnp.max(logits, axis=1, keepdims=True)
    ex = jnp.exp(logits - m)
    p = ex / jnp.sum(ex, axis=1, keepdims=True)

    iota_e = lax.broadcasted_iota(jnp.int32, (BK, E), 1)
    w0 = jnp.max(p, axis=1, keepdims=True)
    i0 = jnp.min(jnp.where(p == w0, iota_e, E), axis=1, keepdims=True)
    p1 = jnp.where(iota_e == i0, -1.0, p)
    w1 = jnp.max(p1, axis=1, keepdims=True)
    i1 = jnp.min(jnp.where(p1 == w1, iota_e, E), axis=1, keepdims=True)
    tot = w0 + w1
    w0n = w0 / tot
    w1n = w1 / tot

    oh0 = (iota_e == i0).astype(jnp.float32)
    oh1 = (iota_e == i1).astype(jnp.float32)
    counts = oh0 + oh1

    r = lax.broadcasted_iota(jnp.int32, (BK, BK), 0)
    c = lax.broadcasted_iota(jnp.int32, (BK, BK), 1)
    ltri = (c < r).astype(jnp.float32)
    carry = carry_ref[...]
    excl = jnp.dot(ltri, counts, preferred_element_type=jnp.float32) + carry
    carry_ref[...] = carry + jnp.sum(counts, axis=0, keepdims=True)

    pos0 = jnp.sum(excl * oh0, axis=1, keepdims=True).astype(jnp.int32)
    pos1 = jnp.sum(excl * oh1, axis=1, keepdims=True).astype(jnp.int32)
    v0 = pos0 < C
    v1 = pos1 < C
    g0 = jnp.where(v0, i0 * C + pos0, S)
    g1 = jnp.where(v1, i1 * C + pos1, S)
    gp_ref[...] = g0 | (g1 << 16)
    w0_ref[...] = jnp.broadcast_to(jnp.where(v0, w0n, 0.0), (BK, 128))
    w1_ref[...] = jnp.broadcast_to(jnp.where(v1, w1n, 0.0), (BK, 128))


def _router(x, wr, br2):
    return pl.pallas_call(
        _router_body,
        grid=(T // BK,),
        in_specs=[
            pl.BlockSpec((BK, D), lambda b: (b, 0)),
            pl.BlockSpec((E, D), lambda b: (0, 0)),
            pl.BlockSpec((1, E), lambda b: (0, 0)),
        ],
        out_specs=(
            pl.BlockSpec((BK, 1), lambda b: (b, 0)),
            pl.BlockSpec((BK, 128), lambda b: (b, 0)),
            pl.BlockSpec((BK, 128), lambda b: (b, 0)),
        ),
        out_shape=(
            jax.ShapeDtypeStruct((T, 1), jnp.int32),
            jax.ShapeDtypeStruct((T, 128), jnp.float32),
            jax.ShapeDtypeStruct((T, 128), jnp.float32),
        ),
        scratch_shapes=[pltpu.VMEM((1, E), jnp.float32)],
    )(x, wr, br2)


@functools.lru_cache(maxsize=None)
def _sc_kernels():
    mesh = plsc.VectorSubcoreMesh(core_axis_name="c", subcore_axis_name="s")

    @functools.partial(
        pl.kernel,
        out_type=(
            jax.ShapeDtypeStruct((S_PAD, D), jnp.float32),
            jax.ShapeDtypeStruct((S_PAD, 128), jnp.float32),
        ),
        mesh=mesh,
        scratch_types=[
            pltpu.VMEM((TPW,), jnp.int32),
            pltpu.VMEM((TPW, D), jnp.float32),
            pltpu.VMEM((TPW, 128), jnp.float32),
            pltpu.VMEM((TPW, 128), jnp.float32),
            pltpu.SemaphoreType.DMA,
        ],
    )
    def dispatch(x_hbm, gp_hbm, w0_hbm, w1_hbm, buf_hbm, wslot_hbm,
                 pk_v, rows_v, wc0_v, wc1_v, sem):
        wid = lax.axis_index("s") * NC + lax.axis_index("c")
        base = wid * TPW
        pltpu.sync_copy(gp_hbm.at[pl.ds(base, TPW)], pk_v)
        pltpu.sync_copy(x_hbm.at[pl.ds(base, TPW)], rows_v)
        pltpu.sync_copy(w0_hbm.at[pl.ds(base, TPW)], wc0_v)
        pltpu.sync_copy(w1_hbm.at[pl.ds(base, TPW)], wc1_v)
        copies = []
        for jj in range(TPW // 16):
            sl = pl.ds(jj * 16, 16)
            pk = pk_v[sl]
            i0 = pk & 0xFFFF
            i1 = pk >> 16
            copies.append(pltpu.async_copy(rows_v.at[sl], buf_hbm.at[i0], sem))
            copies.append(pltpu.async_copy(rows_v.at[sl], buf_hbm.at[i1], sem))
            copies.append(pltpu.async_copy(wc0_v.at[sl], wslot_hbm.at[i0], sem))
            copies.append(pltpu.async_copy(wc1_v.at[sl], wslot_hbm.at[i1], sem))
        for cp in copies:
            cp.wait()

    @functools.partial(
        pl.kernel,
        out_type=jax.ShapeDtypeStruct((T, D), jnp.float32),
        mesh=mesh,
        scratch_types=[
            pltpu.VMEM((TPW,), jnp.int32),
            pltpu.VMEM((TPW, D), jnp.float32),
            pltpu.VMEM((TPW, D), jnp.float32),
            pltpu.SemaphoreType.DMA,
        ],
    )
    def combine(ye_hbm, gp_hbm, out_hbm, pk_v, rows0_v, rows1_v, sem):
        wid = lax.axis_index("s") * NC + lax.axis_index("c")
        base = wid * TPW
        pltpu.sync_copy(gp_hbm.at[pl.ds(base, TPW)], pk_v)
        copies = []
        for jj in range(TPW // 16):
            sl = pl.ds(jj * 16, 16)
            pk = pk_v[sl]
            copies.append(pltpu.async_copy(
                ye_hbm.at[pk & 0xFFFF], rows0_v.at[sl], sem))
            copies.append(pltpu.async_copy(
                ye_hbm.at[pk >> 16], rows1_v.at[sl], sem))
        for cp in copies:
            cp.wait()

        def body(t, carry):
            for j in range(D // 16):
                sl = pl.ds(j * 16, 16)
                rows0_v[t, sl] = rows0_v[t, sl] + rows1_v[t, sl]
            return carry

        lax.fori_loop(0, TPW, body, 0)
        pltpu.sync_copy(rows0_v, out_hbm.at[pl.ds(base, TPW)])

    return dispatch, combine


def _mlp_body(buf_ref, wslot_ref, w13_ref, b13_ref, w2_ref, b2_ref, ye_ref):
    e = pl.program_id(0)

    @pl.when(e < E)
    def _():
        xe = buf_ref[...].astype(jnp.bfloat16)
        h = lax.dot_general(xe, w13_ref[...], (((1,), (1,)), ((), ())),
                            preferred_element_type=jnp.float32) + b13_ref[...]
        gate = jnp.minimum(h[:, :DFF], LIMIT)
        up = jnp.clip(h[:, DFF:], -LIMIT, LIMIT)
        act = gate * (1.0 / (1.0 + jnp.exp(-ALPHA * gate))) * (up + 1.0)
        ye = lax.dot_general(act.astype(jnp.bfloat16), w2_ref[...],
                             (((1,), (1,)), ((), ())),
                             preferred_element_type=jnp.float32) + b2_ref[...]
        ye_ref[...] = ye * wslot_ref[...][:, 0:1]

    @pl.when(e == E)
    def _():
        ye_ref[...] = jnp.zeros((C, D), jnp.float32)


def _mlp(buf, wslot, w13, b13, w2, b2):
    b13 = b13.reshape(E, 1, 2 * DFF)
    b2 = b2.reshape(E, 1, D)
    return pl.pallas_call(
        _mlp_body,
        grid=(E + 1,),
        in_specs=[
            pl.BlockSpec((C, D), lambda e: (jnp.minimum(e, E - 1), 0)),
            pl.BlockSpec((C, 128), lambda e: (jnp.minimum(e, E - 1), 0)),
            pl.BlockSpec((None, 2 * DFF, D), lambda e: (jnp.minimum(e, E - 1), 0, 0)),
            pl.BlockSpec((None, 1, 2 * DFF), lambda e: (jnp.minimum(e, E - 1), 0, 0)),
            pl.BlockSpec((None, D, DFF), lambda e: (jnp.minimum(e, E - 1), 0, 0)),
            pl.BlockSpec((None, 1, D), lambda e: (jnp.minimum(e, E - 1), 0, 0)),
        ],
        out_specs=pl.BlockSpec((C, D), lambda e: (e, 0)),
        out_shape=jax.ShapeDtypeStruct((S + C, D), jnp.float32),
    )(buf, wslot, w13, b13, w2, b2)


def kernel(x, Wr, br, W13, b13, W2, b2):
    br2 = br.reshape(1, E)
    gp, w0b, w1b = _router(x, Wr, br2)
    gp = gp.reshape(T)
    dispatch, combine = _sc_kernels()
    buf, wslot = dispatch(x, gp, w0b, w1b)
    ye = _mlp(buf, wslot, W13.astype(jnp.bfloat16), b13,
              W2.astype(jnp.bfloat16), b2)
    return combine(ye, gp)


# speedup vs baseline: 1.3859x; 1.3859x over previous
import functools

import jax
import jax.numpy as jnp
from jax import lax
from jax.experimental import pallas as pl
from jax.experimental.pallas import tpu as pltpu
from jax.experimental.pallas import tpu_sc as plsc

E = 16
TOP_K = 2
D = 768
DFF = 768
T = 2048
C = 512
S = E * C
S_PAD = S + 8
ALPHA = 1.702
LIMIT = 7.0

NC, NS = 2, 16
NW = NC * NS
TPW = T // NW


BK = 512


def _router_body(x_ref, wr_ref, br_ref, gp_ref, w0_ref, w1_ref, carry_ref):
    b = pl.program_id(0)

    @pl.when(b == 0)
    def _():
        carry_ref[...] = jnp.zeros((1, E), jnp.float32)

    logits = lax.dot_general(x_ref[...], wr_ref[...], (((1,), (1,)), ((), ())),
                             preferred_element_type=jnp.float32) + br_ref[...]
    m = jnp.max(logits, axis=1, keepdims=True)
    ex = jnp.exp(logits - m)
    p = ex / jnp.sum(ex, axis=1, keepdims=True)

    iota_e = lax.broadcasted_iota(jnp.int32, (BK, E), 1)
    w0 = jnp.max(p, axis=1, keepdims=True)
    i0 = jnp.min(jnp.where(p == w0, iota_e, E), axis=1, keepdims=True)
    p1 = jnp.where(iota_e == i0, -1.0, p)
    w1 = jnp.max(p1, axis=1, keepdims=True)
    i1 = jnp.min(jnp.where(p1 == w1, iota_e, E), axis=1, keepdims=True)
    tot = w0 + w1
    w0n = w0 / tot
    w1n = w1 / tot

    oh0 = (iota_e == i0).astype(jnp.float32)
    oh1 = (iota_e == i1).astype(jnp.float32)
    counts = oh0 + oh1

    r = lax.broadcasted_iota(jnp.int32, (BK, BK), 0)
    c = lax.broadcasted_iota(jnp.int32, (BK, BK), 1)
    ltri = (c < r).astype(jnp.float32)
    carry = carry_ref[...]
    excl = jnp.dot(ltri, counts, preferred_element_type=jnp.float32) + carry
    carry_ref[...] = carry + jnp.sum(counts, axis=0, keepdims=True)

    pos0 = jnp.sum(excl * oh0, axis=1, keepdims=True).astype(jnp.int32)
    pos1 = jnp.sum(excl * oh1, axis=1, keepdims=True).astype(jnp.int32)
    v0 = pos0 < C
    v1 = pos1 < C
    g0 = jnp.where(v0, i0 * C + pos0, S)
    g1 = jnp.where(v1, i1 * C + pos1, S)
    gp_ref[...] = g0 | (g1 << 16)
    w0_ref[...] = jnp.broadcast_to(jnp.where(v0, w0n, 0.0), (BK, 128))
    w1_ref[...] = jnp.broadcast_to(jnp.where(v1, w1n, 0.0), (BK, 128))


def _router(x, wr, br2):
    return pl.pallas_call(
        _router_body,
        grid=(T // BK,),
        in_specs=[
            pl.BlockSpec((BK, D), lambda b: (b, 0)),
            pl.BlockSpec((E, D), lambda b: (0, 0)),
            pl.BlockSpec((1, E), lambda b: (0, 0)),
        ],
        out_specs=(
            pl.BlockSpec((BK, 1), lambda b: (b, 0)),
            pl.BlockSpec((BK, 128), lambda b: (b, 0)),
            pl.BlockSpec((BK, 128), lambda b: (b, 0)),
        ),
        out_shape=(
            jax.ShapeDtypeStruct((T, 1), jnp.int32),
            jax.ShapeDtypeStruct((T, 128), jnp.float32),
            jax.ShapeDtypeStruct((T, 128), jnp.float32),
        ),
        scratch_shapes=[pltpu.VMEM((1, E), jnp.float32)],
    )(x, wr, br2)


@functools.lru_cache(maxsize=None)
def _sc_kernels():
    mesh = plsc.VectorSubcoreMesh(core_axis_name="c", subcore_axis_name="s")

    @functools.partial(
        pl.kernel,
        out_type=(
            jax.ShapeDtypeStruct((S_PAD, D), jnp.float32),
            jax.ShapeDtypeStruct((S_PAD, 128), jnp.float32),
        ),
        mesh=mesh,
        scratch_types=[
            pltpu.VMEM((TPW,), jnp.int32),
            pltpu.VMEM((TPW, D), jnp.float32),
            pltpu.VMEM((TPW, 128), jnp.float32),
            pltpu.VMEM((TPW, 128), jnp.float32),
            pltpu.SemaphoreType.DMA,
        ],
    )
    def dispatch(x_hbm, gp_hbm, w0_hbm, w1_hbm, buf_hbm, wslot_hbm,
                 pk_v, rows_v, wc0_v, wc1_v, sem):
        wid = lax.axis_index("s") * NC + lax.axis_index("c")
        base = wid * TPW
        pltpu.sync_copy(gp_hbm.at[pl.ds(base, TPW)], pk_v)
        pltpu.sync_copy(x_hbm.at[pl.ds(base, TPW)], rows_v)
        pltpu.sync_copy(w0_hbm.at[pl.ds(base, TPW)], wc0_v)
        pltpu.sync_copy(w1_hbm.at[pl.ds(base, TPW)], wc1_v)
        copies = []
        for jj in range(TPW // 16):
            sl = pl.ds(jj * 16, 16)
            pk = pk_v[sl]
            i0 = pk & 0xFFFF
            i1 = pk >> 16
            copies.append(pltpu.async_copy(rows_v.at[sl], buf_hbm.at[i0], sem))
            copies.append(pltpu.async_copy(rows_v.at[sl], buf_hbm.at[i1], sem))
            copies.append(pltpu.async_copy(wc0_v.at[sl], wslot_hbm.at[i0], sem))
            copies.append(pltpu.async_copy(wc1_v.at[sl], wslot_hbm.at[i1], sem))
        for cp in copies:
            cp.wait()

    @functools.partial(
        pl.kernel,
        out_type=jax.ShapeDtypeStruct((T, D), jnp.float32),
        mesh=mesh,
        scratch_types=[
            pltpu.VMEM((TPW,), jnp.int32),
            pltpu.VMEM((TPW, D), jnp.float32),
            pltpu.VMEM((TPW, D), jnp.float32),
            pltpu.SemaphoreType.DMA,
        ],
    )
    def combine(ye_hbm, gp_hbm, out_hbm, pk_v, rows0_v, rows1_v, sem):
        wid = lax.axis_index("s") * NC + lax.axis_index("c")
        base = wid * TPW
        pltpu.sync_copy(gp_hbm.at[pl.ds(base, TPW)], pk_v)
        copies = []
        for jj in range(TPW // 16):
            sl = pl.ds(jj * 16, 16)
            pk = pk_v[sl]
            copies.append(pltpu.async_copy(
                ye_hbm.at[pk & 0xFFFF], rows0_v.at[sl], sem))
            copies.append(pltpu.async_copy(
                ye_hbm.at[pk >> 16], rows1_v.at[sl], sem))
        for cp in copies:
            cp.wait()

        def body(t, carry):
            for j in range(D // 16):
                sl = pl.ds(j * 16, 16)
                rows0_v[t, sl] = rows0_v[t, sl] + rows1_v[t, sl]
            return carry

        lax.fori_loop(0, TPW, body, 0)
        pltpu.sync_copy(rows0_v, out_hbm.at[pl.ds(base, TPW)])

    return dispatch, combine


def _mlp_body(buf_ref, wslot_ref, w13_ref, b13_ref, w2_ref, b2_ref, ye_ref):
    e = pl.program_id(0)

    @pl.when(e < E)
    def _():
        xe = buf_ref[...]
        h = lax.dot_general(xe, w13_ref[...], (((1,), (1,)), ((), ())),
                            preferred_element_type=jnp.float32) + b13_ref[...]
        gate = jnp.minimum(h[:, :DFF], LIMIT)
        up = jnp.clip(h[:, DFF:], -LIMIT, LIMIT)
        act = gate * (1.0 / (1.0 + jnp.exp(-ALPHA * gate))) * (up + 1.0)
        ye = lax.dot_general(act, w2_ref[...], (((1,), (1,)), ((), ())),
                             preferred_element_type=jnp.float32) + b2_ref[...]
        ye_ref[...] = ye * wslot_ref[...][:, 0:1]

    @pl.when(e == E)
    def _():
        ye_ref[...] = jnp.zeros((C, D), jnp.float32)


def _mlp(buf, wslot, w13, b13, w2, b2):
    b13 = b13.reshape(E, 1, 2 * DFF)
    b2 = b2.reshape(E, 1, D)
    return pl.pallas_call(
        _mlp_body,
        grid=(E + 1,),
        in_specs=[
            pl.BlockSpec((C, D), lambda e: (jnp.minimum(e, E - 1), 0)),
            pl.BlockSpec((C, 128), lambda e: (jnp.minimum(e, E - 1), 0)),
            pl.BlockSpec((None, 2 * DFF, D), lambda e: (jnp.minimum(e, E - 1), 0, 0)),
            pl.BlockSpec((None, 1, 2 * DFF), lambda e: (jnp.minimum(e, E - 1), 0, 0)),
            pl.BlockSpec((None, D, DFF), lambda e: (jnp.minimum(e, E - 1), 0, 0)),
            pl.BlockSpec((None, 1, D), lambda e: (jnp.minimum(e, E - 1), 0, 0)),
        ],
        out_specs=pl.BlockSpec((C, D), lambda e: (e, 0)),
        out_shape=jax.ShapeDtypeStruct((S + C, D), jnp.float32),
    )(buf, wslot, w13, b13, w2, b2)


def kernel(x, Wr, br, W13, b13, W2, b2):
    br2 = br.reshape(1, E)
    gp, w0b, w1b = _router(x, Wr, br2)
    gp = gp.reshape(T)
    dispatch, combine = _sc_kernels()
    buf, wslot = dispatch(x, gp, w0b, w1b)
    ye = _mlp(buf, wslot, W13, b13, W2, b2)
    return combine(ye, gp)


# speedup vs baseline: 1.4020x; 1.0117x over previous
import functools

import jax
import jax.numpy as jnp
from jax import lax
from jax.experimental import pallas as pl
from jax.experimental.pallas import tpu as pltpu
from jax.experimental.pallas import tpu_sc as plsc

E = 16
TOP_K = 2
D = 768
DFF = 768
T = 2048
C = 512
S = E * C
S_PAD = S + 8
ALPHA = 1.702
LIMIT = 7.0

NC, NS = 2, 16
NW = NC * NS
TPW = T // NW


BK = 512


def _router_body(x_ref, wr_ref, br_ref, gp_ref, w0_ref, w1_ref, carry_ref):
    b = pl.program_id(0)

    @pl.when(b == 0)
    def _():
        carry_ref[...] = jnp.zeros((1, E), jnp.float32)

    logits = lax.dot_general(x_ref[...], wr_ref[...], (((1,), (1,)), ((), ())),
                             preferred_element_type=jnp.float32) + br_ref[...]
    m = jnp.max(logits, axis=1, keepdims=True)
    ex = jnp.exp(logits - m)
    p = ex / jnp.sum(ex, axis=1, keepdims=True)

    iota_e = lax.broadcasted_iota(jnp.int32, (BK, E), 1)
    w0 = jnp.max(p, axis=1, keepdims=True)
    i0 = jnp.min(jnp.where(p == w0, iota_e, E), axis=1, keepdims=True)
    p1 = jnp.where(iota_e == i0, -1.0, p)
    w1 = jnp.max(p1, axis=1, keepdims=True)
    i1 = jnp.min(jnp.where(p1 == w1, iota_e, E), axis=1, keepdims=True)
    tot = w0 + w1
    w0n = w0 / tot
    w1n = w1 / tot

    oh0 = (iota_e == i0).astype(jnp.float32)
    oh1 = (iota_e == i1).astype(jnp.float32)
    counts = oh0 + oh1

    r = lax.broadcasted_iota(jnp.int32, (BK, BK), 0)
    c = lax.broadcasted_iota(jnp.int32, (BK, BK), 1)
    ltri = (c < r).astype(jnp.float32)
    carry = carry_ref[...]
    excl = jnp.dot(ltri, counts, preferred_element_type=jnp.float32) + carry
    carry_ref[...] = carry + jnp.sum(counts, axis=0, keepdims=True)

    pos0 = jnp.sum(excl * oh0, axis=1, keepdims=True).astype(jnp.int32)
    pos1 = jnp.sum(excl * oh1, axis=1, keepdims=True).astype(jnp.int32)
    v0 = pos0 < C
    v1 = pos1 < C
    g0 = jnp.where(v0, i0 * C + pos0, S)
    g1 = jnp.where(v1, i1 * C + pos1, S)
    gp_ref[...] = g0 | (g1 << 16)
    w0_ref[...] = jnp.broadcast_to(jnp.where(v0, w0n, 0.0), (BK, 128))
    w1_ref[...] = jnp.broadcast_to(jnp.where(v1, w1n, 0.0), (BK, 128))


def _router(x, wr, br2):
    return pl.pallas_call(
        _router_body,
        grid=(T // BK,),
        in_specs=[
            pl.BlockSpec((BK, D), lambda b: (b, 0)),
            pl.BlockSpec((E, D), lambda b: (0, 0)),
            pl.BlockSpec((1, E), lambda b: (0, 0)),
        ],
        out_specs=(
            pl.BlockSpec((BK, 1), lambda b: (b, 0)),
            pl.BlockSpec((BK, 128), lambda b: (b, 0)),
            pl.BlockSpec((BK, 128), lambda b: (b, 0)),
        ),
        out_shape=(
            jax.ShapeDtypeStruct((T, 1), jnp.int32),
            jax.ShapeDtypeStruct((T, 128), jnp.float32),
            jax.ShapeDtypeStruct((T, 128), jnp.float32),
        ),
        scratch_shapes=[pltpu.VMEM((1, E), jnp.float32)],
    )(x, wr, br2)


@functools.lru_cache(maxsize=None)
def _sc_kernels():
    mesh = plsc.VectorSubcoreMesh(core_axis_name="c", subcore_axis_name="s")

    @functools.partial(
        pl.kernel,
        out_type=jax.ShapeDtypeStruct((S_PAD, D), jnp.float32),
        mesh=mesh,
        scratch_types=[
            pltpu.VMEM((TPW,), jnp.int32),
            pltpu.VMEM((TPW, D), jnp.float32),
            pltpu.SemaphoreType.DMA,
        ],
    )
    def dispatch(x_hbm, gp_hbm, buf_hbm, pk_v, rows_v, sem):
        wid = lax.axis_index("s") * NC + lax.axis_index("c")
        base = wid * TPW
        pltpu.sync_copy(gp_hbm.at[pl.ds(base, TPW)], pk_v)
        pltpu.sync_copy(x_hbm.at[pl.ds(base, TPW)], rows_v)
        copies = []
        for jj in range(TPW // 16):
            sl = pl.ds(jj * 16, 16)
            pk = pk_v[sl]
            i0 = pk & 0xFFFF
            i1 = pk >> 16
            copies.append(pltpu.async_copy(rows_v.at[sl], buf_hbm.at[i0], sem))
            copies.append(pltpu.async_copy(rows_v.at[sl], buf_hbm.at[i1], sem))
        for cp in copies:
            cp.wait()

    @functools.partial(
        pl.kernel,
        out_type=jax.ShapeDtypeStruct((T, D), jnp.float32),
        mesh=mesh,
        scratch_types=[
            pltpu.VMEM((TPW,), jnp.int32),
            pltpu.VMEM((TPW, D), jnp.float32),
            pltpu.VMEM((TPW, D), jnp.float32),
            pltpu.VMEM((TPW, 128), jnp.float32),
            pltpu.VMEM((TPW, 128), jnp.float32),
            pltpu.SemaphoreType.DMA,
        ],
    )
    def combine(ye_hbm, gp_hbm, w0_hbm, w1_hbm, out_hbm,
                pk_v, rows0_v, rows1_v, wc0_v, wc1_v, sem):
        wid = lax.axis_index("s") * NC + lax.axis_index("c")
        base = wid * TPW
        pltpu.sync_copy(gp_hbm.at[pl.ds(base, TPW)], pk_v)
        pltpu.sync_copy(w0_hbm.at[pl.ds(base, TPW)], wc0_v)
        pltpu.sync_copy(w1_hbm.at[pl.ds(base, TPW)], wc1_v)
        copies = []
        for jj in range(TPW // 16):
            sl = pl.ds(jj * 16, 16)
            pk = pk_v[sl]
            copies.append(pltpu.async_copy(
                ye_hbm.at[pk & 0xFFFF], rows0_v.at[sl], sem))
            copies.append(pltpu.async_copy(
                ye_hbm.at[pk >> 16], rows1_v.at[sl], sem))
        for cp in copies:
            cp.wait()

        def body(t, carry):
            w0v = wc0_v[t, pl.ds(0, 16)]
            w1v = wc1_v[t, pl.ds(0, 16)]
            for j in range(D // 16):
                sl = pl.ds(j * 16, 16)
                rows0_v[t, sl] = rows0_v[t, sl] * w0v + rows1_v[t, sl] * w1v
            return carry

        lax.fori_loop(0, TPW, body, 0)
        pltpu.sync_copy(rows0_v, out_hbm.at[pl.ds(base, TPW)])

    return dispatch, combine


def _mlp_body(buf_ref, w13_ref, b13_ref, w2_ref, b2_ref, ye_ref):
    e = pl.program_id(0)

    @pl.when(e < E)
    def _():
        xe = buf_ref[...]
        h = lax.dot_general(xe, w13_ref[...], (((1,), (1,)), ((), ())),
                            preferred_element_type=jnp.float32) + b13_ref[...]
        gate = jnp.minimum(h[:, :DFF], LIMIT)
        up = jnp.clip(h[:, DFF:], -LIMIT, LIMIT)
        act = gate * (1.0 / (1.0 + jnp.exp(-ALPHA * gate))) * (up + 1.0)
        ye = lax.dot_general(act, w2_ref[...], (((1,), (1,)), ((), ())),
                             preferred_element_type=jnp.float32) + b2_ref[...]
        ye_ref[...] = ye

    @pl.when(e == E)
    def _():
        ye_ref[...] = jnp.zeros((C, D), jnp.float32)


def _mlp(buf, w13, b13, w2, b2):
    b13 = b13.reshape(E, 1, 2 * DFF)
    b2 = b2.reshape(E, 1, D)
    return pl.pallas_call(
        _mlp_body,
        grid=(E + 1,),
        in_specs=[
            pl.BlockSpec((C, D), lambda e: (jnp.minimum(e, E - 1), 0)),
            pl.BlockSpec((None, 2 * DFF, D), lambda e: (jnp.minimum(e, E - 1), 0, 0)),
            pl.BlockSpec((None, 1, 2 * DFF), lambda e: (jnp.minimum(e, E - 1), 0, 0)),
            pl.BlockSpec((None, D, DFF), lambda e: (jnp.minimum(e, E - 1), 0, 0)),
            pl.BlockSpec((None, 1, D), lambda e: (jnp.minimum(e, E - 1), 0, 0)),
        ],
        out_specs=pl.BlockSpec((C, D), lambda e: (e, 0)),
        out_shape=jax.ShapeDtypeStruct((S + C, D), jnp.float32),
    )(buf, w13, b13, w2, b2)


def kernel(x, Wr, br, W13, b13, W2, b2):
    br2 = br.reshape(1, E)
    gp, w0b, w1b = _router(x, Wr, br2)
    gp = gp.reshape(T)
    dispatch, combine = _sc_kernels()
    buf = dispatch(x, gp)
    ye = _mlp(buf, W13, b13, W2, b2)
    return combine(ye, gp, w0b, w1b)
